# baseline (speedup 1.0000x reference)
import functools

import jax
import jax.numpy as jnp
from jax import lax
from jax.experimental import pallas as pl
from jax.experimental.pallas import tpu as pltpu
from jax.experimental.pallas import tpu_sc as plsc

_BATCH = 16384
_Z_DIM = 64
_NUM_WORKERS = 32
_B_PER_W = _BATCH // _NUM_WORKERS


def kernel(u, embed_mean, embed_log_var):
    idx = u.astype(jnp.int32)
    out_sds = jax.ShapeDtypeStruct((_BATCH, _Z_DIM), embed_mean.dtype)
    mesh = plsc.VectorSubcoreMesh(core_axis_name="c", subcore_axis_name="s")

    @jax.jit
    @functools.partial(
        pl.kernel,
        out_type=(out_sds, out_sds),
        mesh=mesh,
        compiler_params=pltpu.CompilerParams(skip_device_barrier=True),
        scratch_types=[
            pltpu.VMEM((_B_PER_W,), jnp.int32),
            pltpu.SemaphoreType.DMA,
            pltpu.SemaphoreType.DMA,
            pltpu.SemaphoreType.DMA,
        ],
    )
    def _gather(mean_hbm, logvar_hbm, idx_hbm, om_hbm, ov_hbm,
                idx_v, sem_i, sem_m, sem_v):
        wid = lax.axis_index("s") * 2 + lax.axis_index("c")
        base = wid * _B_PER_W
        pltpu.async_copy(idx_hbm.at[pl.ds(base, _B_PER_W)], idx_v, sem_i).wait()

        @pl.loop(0, _B_PER_W // 16)
        def _(g):
            vec = idx_v[pl.ds(g * 16, 16)]
            for j in range(16):
                row = vec[j]
                i = g * 16 + j
                pltpu.async_copy(mean_hbm.at[row], om_hbm.at[base + i], sem_m)
                pltpu.async_copy(
                    logvar_hbm.at[row], ov_hbm.at[base + i], sem_v)

        pltpu.make_async_copy(
            mean_hbm.at[pl.ds(0, _B_PER_W)],
            om_hbm.at[pl.ds(0, _B_PER_W)], sem_m).wait()
        pltpu.make_async_copy(
            logvar_hbm.at[pl.ds(0, _B_PER_W)],
            ov_hbm.at[pl.ds(0, _B_PER_W)], sem_v).wait()

    return _gather(embed_mean, embed_log_var, idx)


# speedup vs baseline: 5.2112x; 5.2112x over previous
import functools

import jax
import jax.numpy as jnp
from jax import lax
from jax.experimental import pallas as pl
from jax.experimental.pallas import tpu as pltpu
from jax.experimental.pallas import tpu_sc as plsc

_BATCH = 16384
_Z_DIM = 64
_NUM_WORKERS = 32
_B_PER_W = _BATCH // _NUM_WORKERS
_CHUNK = 128
_N_CHUNKS = _B_PER_W // _CHUNK


def kernel(u, embed_mean, embed_log_var):
    idx = u.astype(jnp.int32)
    out_sds = jax.ShapeDtypeStruct((_BATCH, _Z_DIM), embed_mean.dtype)
    mesh = plsc.VectorSubcoreMesh(core_axis_name="c", subcore_axis_name="s")

    @jax.jit
    @functools.partial(
        pl.kernel,
        out_type=(out_sds, out_sds),
        mesh=mesh,
        compiler_params=pltpu.CompilerParams(skip_device_barrier=True),
        scratch_types=[
            pltpu.VMEM((_B_PER_W,), jnp.int32),
            [pltpu.VMEM((_CHUNK, _Z_DIM), jnp.float32) for _ in range(2)],
            [pltpu.VMEM((_CHUNK, _Z_DIM), jnp.float32) for _ in range(2)],
            pltpu.SemaphoreType.DMA,
            [pltpu.SemaphoreType.DMA for _ in range(2)],
            [pltpu.SemaphoreType.DMA for _ in range(2)],
            [pltpu.SemaphoreType.DMA for _ in range(2)],
            [pltpu.SemaphoreType.DMA for _ in range(2)],
        ],
    )
    def _gather(mean_hbm, logvar_hbm, idx_hbm, om_hbm, ov_hbm,
                idx_v, mbuf, vbuf, sem_i, sem_m, sem_v, sem_wm, sem_wv):
        wid = lax.axis_index("s") * 2 + lax.axis_index("c")
        base = wid * _B_PER_W
        pltpu.async_copy(idx_hbm.at[pl.ds(base, _B_PER_W)], idx_v, sem_i).wait()

        def drain_and_writeback(k):
            b = k % 2
            pltpu.make_async_copy(
                mean_hbm.at[pl.ds(0, _CHUNK)], mbuf[b], sem_m[b]).wait()
            pltpu.make_async_copy(
                logvar_hbm.at[pl.ds(0, _CHUNK)], vbuf[b], sem_v[b]).wait()
            out_slc = pl.ds(base + k * _CHUNK, _CHUNK)
            pltpu.async_copy(mbuf[b], om_hbm.at[out_slc], sem_wm[b])
            pltpu.async_copy(vbuf[b], ov_hbm.at[out_slc], sem_wv[b])

        for k in range(_N_CHUNKS):
            b = k % 2
            cbase = k * _CHUNK
            if k >= 2:
                pltpu.make_async_copy(
                    mbuf[b], om_hbm.at[pl.ds(0, _CHUNK)], sem_wm[b]).wait()
                pltpu.make_async_copy(
                    vbuf[b], ov_hbm.at[pl.ds(0, _CHUNK)], sem_wv[b]).wait()

            @pl.loop(0, _CHUNK // 16)
            def _(g):
                vec = idx_v[pl.ds(cbase + g * 16, 16)]
                for j in range(16):
                    row = vec[j]
                    i = g * 16 + j
                    pltpu.async_copy(
                        mean_hbm.at[row], mbuf[b].at[i], sem_m[b])
                    pltpu.async_copy(
                        logvar_hbm.at[row], vbuf[b].at[i], sem_v[b])

            if k >= 1:
                drain_and_writeback(k - 1)

        drain_and_writeback(_N_CHUNKS - 1)
        for b in range(2):
            pltpu.make_async_copy(
                mbuf[b], om_hbm.at[pl.ds(0, _CHUNK)], sem_wm[b]).wait()
            pltpu.make_async_copy(
                vbuf[b], ov_hbm.at[pl.ds(0, _CHUNK)], sem_wv[b]).wait()

    return _gather(embed_mean, embed_log_var, idx)
